# baseline (speedup 1.0000x reference)
import functools

import jax
import jax.numpy as jnp
from jax import lax
from jax.experimental import pallas as pl
from jax.experimental.pallas import tpu as pltpu
from jax.experimental.pallas import tpu_sc as plsc

N = 10000
D = 128
HD = D // 2
NP = 10240
E = 320000
NC = 2
NS = 16
CHUNK = 128
GROUPS = 79
GMAX = GROUPS
SLOWC = 0
G_SLOW = 56
STEAL = GROUPS - G_SLOW
T_EDGES = GROUPS * CHUNK
EPAD = NC * NS * T_EDGES
ROWS_PER_TILE = NP // NS
R_BLK = 1024


def _sc_mesh():
    return plsc.VectorSubcoreMesh(core_axis_name="c", subcore_axis_name="s",
                                  num_cores=NC, num_subcores=NS)


def _deg_body(src_hbm, dst_hbm, out_hbm, src_idx, dst_idx,
              hist_s, hist_d, sem):
    c = lax.axis_index("c")
    s = lax.axis_index("s")
    wid = c * NS + s
    pltpu.sync_copy(src_hbm.at[wid], src_idx)
    pltpu.sync_copy(dst_hbm.at[wid], dst_idx)
    zero16 = jnp.zeros((16,), jnp.float32)
    ones = jnp.full((16,), 1.0, jnp.float32)

    def zero(i, _):
        hist_s[pl.ds(i * 16, 16)] = zero16
        hist_d[pl.ds(i * 16, 16)] = zero16
        return 0

    lax.fori_loop(0, NP // 16, zero, 0)

    def step(g, _):
        for j in range(CHUNK // 16):
            plsc.addupdate_scatter(hist_s, [src_idx[g, pl.ds(j * 16, 16)]], ones)
            plsc.addupdate_scatter(hist_d, [dst_idx[g, pl.ds(j * 16, 16)]], ones)
        return 0

    lax.fori_loop(0, GROUPS, step, 0)
    pltpu.sync_copy(hist_s, out_hbm.at[wid, 0])
    pltpu.sync_copy(hist_d, out_hbm.at[wid, 1])


def _sc_degrees(src3d, dst3d):
    return pl.kernel(
        _deg_body,
        out_type=jax.ShapeDtypeStruct((NC * NS, 2, NP), jnp.float32),
        mesh=_sc_mesh(),
        scratch_types=[
            pltpu.VMEM((GROUPS, CHUNK), jnp.int32),
            pltpu.VMEM((GROUPS, CHUNK), jnp.int32),
            pltpu.VMEM((NP,), jnp.float32),
            pltpu.VMEM((NP,), jnp.float32),
            pltpu.SemaphoreType.DMA,
        ],
        compiler_params=pltpu.CompilerParams(needs_layout_passes=False),
    )(src3d, dst3d)


def _agg_body(src_hbm, dst_hbm, xs_hbm, zrows_hbm, out_hbm,
              src_idx, dst_idx, src_idx2, dst_idx2, rows, acc, sem):
    c = lax.axis_index("c")
    s = lax.axis_index("s")
    wid = c * NS + s
    rbase = s * ROWS_PER_TILE

    pwid = SLOWC * NS + s
    main_bound = jnp.where(c == SLOWC, G_SLOW, GROUPS)
    steal_bound = jnp.where(c == SLOWC, 0, STEAL)

    pltpu.sync_copy(src_hbm.at[wid], src_idx)
    pltpu.sync_copy(dst_hbm.at[wid], dst_idx)
    pltpu.sync_copy(src_hbm.at[pwid, pl.ds(G_SLOW, STEAL)], src_idx2)
    pltpu.sync_copy(dst_hbm.at[pwid, pl.ds(G_SLOW, STEAL)], dst_idx2)
    pltpu.sync_copy(zrows_hbm, acc.at[pl.ds(rbase, ROWS_PER_TILE)])
    plsc.subcore_barrier()

    def step(g, _):
        pltpu.async_copy(xs_hbm.at[src_idx.at[g]], rows, sem).wait()
        pltpu.sync_copy(rows, acc.at[dst_idx.at[g]], add=True)
        return 0

    def step2(g, _):
        pltpu.async_copy(xs_hbm.at[src_idx2.at[g]], rows, sem).wait()
        pltpu.sync_copy(rows, acc.at[dst_idx2.at[g]], add=True)
        return 0

    lax.fori_loop(0, main_bound, step, 0)
    lax.fori_loop(0, steal_bound, step2, 0)
    plsc.subcore_barrier()
    pltpu.sync_copy(acc.at[pl.ds(rbase, ROWS_PER_TILE)],
                    out_hbm.at[c, pl.ds(rbase, ROWS_PER_TILE)])


def _sc_aggregate(src3d, dst3d, xs, zrows):
    return pl.kernel(
        _agg_body,
        out_type=jax.ShapeDtypeStruct((NC, NP, D), jnp.float32),
        mesh=_sc_mesh(),
        scratch_types=[
            pltpu.VMEM((GMAX, CHUNK), jnp.int32),
            pltpu.VMEM((GMAX, CHUNK), jnp.int32),
            pltpu.VMEM((STEAL, CHUNK), jnp.int32),
            pltpu.VMEM((STEAL, CHUNK), jnp.int32),
            pltpu.VMEM((CHUNK, D), jnp.float32),
            pltpu.VMEM_SHARED((NP, D), jnp.float32),
            pltpu.SemaphoreType.DMA,
        ],
    )(src3d, dst3d, xs, zrows)


def _red_body(deg_ref, out_ref):
    out_ref[...] = jnp.sum(deg_ref[...], axis=0)


def _tc_reduce(deg):
    return pl.pallas_call(
        _red_body,
        in_specs=[pl.BlockSpec((NC * NS, 2, NP), lambda: (0, 0, 0))],
        out_specs=pl.BlockSpec((2, NP), lambda: (0, 0)),
        out_shape=jax.ShapeDtypeStruct((2, NP), jnp.float32),
    )(deg)


def _pre_body(deg_ref, h_ref, xs_ref, ns_ref, nd_ref):
    i = pl.program_id(0)
    d = deg_ref[...]
    row = i * R_BLK + lax.broadcasted_iota(jnp.int32, (R_BLK, 1), 0)
    valid = row < N
    ns = jnp.where(valid, lax.rsqrt(jnp.maximum(d[0], 1.0)), 0.0)
    nd = jnp.where(valid, lax.rsqrt(jnp.maximum(d[1], 1.0)), 0.0)
    ns_ref[...] = ns
    nd_ref[...] = nd
    xs_ref[...] = h_ref[...] * ns


def _tc_pre(deg3, h_pad):
    grid = NP // R_BLK
    return pl.pallas_call(
        _pre_body,
        grid=(grid,),
        in_specs=[
            pl.BlockSpec((2, R_BLK, 1), lambda i: (0, i, 0)),
            pl.BlockSpec((R_BLK, D), lambda i: (i, 0)),
        ],
        out_specs=[
            pl.BlockSpec((R_BLK, D), lambda i: (i, 0)),
            pl.BlockSpec((R_BLK, 1), lambda i: (i, 0)),
            pl.BlockSpec((R_BLK, 1), lambda i: (i, 0)),
        ],
        out_shape=[
            jax.ShapeDtypeStruct((NP, D), jnp.float32),
            jax.ShapeDtypeStruct((NP, 1), jnp.float32),
            jax.ShapeDtypeStruct((NP, 1), jnp.float32),
        ],
    )(deg3, h_pad)


def _layer_body(agg_ref, nd_ref, ns_ref, w_ref, b_ref, out_ref, *, relu, scale):
    a = agg_ref[0] + agg_ref[1]
    y = jnp.dot(a * nd_ref[...], w_ref[...],
                preferred_element_type=jnp.float32) + b_ref[...]
    if relu:
        y = jnp.maximum(y, 0.0)
    if scale:
        y = y * ns_ref[...]
    out_ref[...] = y


def _tc_layer(agg, nd, ns, w, b2d, relu, scale):
    grid = NP // R_BLK
    return pl.pallas_call(
        functools.partial(_layer_body, relu=relu, scale=scale),
        grid=(grid,),
        in_specs=[
            pl.BlockSpec((NC, R_BLK, D), lambda i: (0, i, 0)),
            pl.BlockSpec((R_BLK, 1), lambda i: (i, 0)),
            pl.BlockSpec((R_BLK, 1), lambda i: (i, 0)),
            pl.BlockSpec((D, D), lambda i: (0, 0)),
            pl.BlockSpec((1, D), lambda i: (0, 0)),
        ],
        out_specs=pl.BlockSpec((R_BLK, D), lambda i: (i, 0)),
        out_shape=jax.ShapeDtypeStruct((NP, D), jnp.float32),
    )(agg, nd, ns, w, b2d)


def kernel(h, edge_index, W1, b1, W2, b2, W3, b3):
    e32 = edge_index.astype(jnp.int32)
    pad = jnp.full((EPAD - E,), N, dtype=jnp.int32)
    src2d = jnp.concatenate([e32[0], pad]).reshape(NC * NS, GROUPS, CHUNK)
    dst2d = jnp.concatenate([e32[1], pad]).reshape(NC * NS, GROUPS, CHUNK)
    h_pad = jnp.zeros((NP, D), jnp.float32).at[:N].set(h)
    zrows = jnp.zeros((ROWS_PER_TILE, D), jnp.float32)

    deg = _tc_reduce(_sc_degrees(src2d, dst2d))
    xs, ns, nd = _tc_pre(deg.reshape(2, NP, 1), h_pad)
    agg = _sc_aggregate(src2d, dst2d, xs, zrows)
    xs = _tc_layer(agg, nd, ns, W1, b1.reshape(1, D), True, True)
    agg = _sc_aggregate(src2d, dst2d, xs, zrows)
    xs = _tc_layer(agg, nd, ns, W2, b2.reshape(1, D), True, True)
    agg = _sc_aggregate(src2d, dst2d, xs, zrows)
    out = _tc_layer(agg, nd, ns, W3, b3.reshape(1, D), False, False)
    return out[:N]


# speedup vs baseline: 1.1159x; 1.1159x over previous
import functools

import jax
import jax.numpy as jnp
from jax import lax
from jax.experimental import pallas as pl
from jax.experimental.pallas import tpu as pltpu
from jax.experimental.pallas import tpu_sc as plsc

N = 10000
D = 128
HD = D // 2
NP = 10240
E = 320000
NC = 2
NS = 16
CHUNK = 128
GROUPS = 79
GMAX = GROUPS
T_EDGES = GROUPS * CHUNK
EPAD = NC * NS * T_EDGES
ROWS_PER_TILE = NP // NS
R_BLK = 1024


def _sc_mesh():
    return plsc.VectorSubcoreMesh(core_axis_name="c", subcore_axis_name="s",
                                  num_cores=NC, num_subcores=NS)


def _deg_body(src_hbm, dst_hbm, out_hbm, src_idx, dst_idx,
              hist_s, hist_d, sem):
    c = lax.axis_index("c")
    s = lax.axis_index("s")
    wid = c * NS + s
    pltpu.sync_copy(src_hbm.at[wid], src_idx)
    pltpu.sync_copy(dst_hbm.at[wid], dst_idx)
    zero16 = jnp.zeros((16,), jnp.float32)
    ones = jnp.full((16,), 1.0, jnp.float32)

    def zero(i, _):
        hist_s[pl.ds(i * 16, 16)] = zero16
        hist_d[pl.ds(i * 16, 16)] = zero16
        return 0

    lax.fori_loop(0, NP // 16, zero, 0)

    def step(g, _):
        for j in range(CHUNK // 16):
            plsc.addupdate_scatter(hist_s, [src_idx[g, pl.ds(j * 16, 16)]], ones)
            plsc.addupdate_scatter(hist_d, [dst_idx[g, pl.ds(j * 16, 16)]], ones)
        return 0

    lax.fori_loop(0, GROUPS, step, 0)
    pltpu.sync_copy(hist_s, out_hbm.at[wid, 0])
    pltpu.sync_copy(hist_d, out_hbm.at[wid, 1])


def _sc_degrees(src3d, dst3d):
    return pl.kernel(
        _deg_body,
        out_type=jax.ShapeDtypeStruct((NC * NS, 2, NP), jnp.float32),
        mesh=_sc_mesh(),
        scratch_types=[
            pltpu.VMEM((GROUPS, CHUNK), jnp.int32),
            pltpu.VMEM((GROUPS, CHUNK), jnp.int32),
            pltpu.VMEM((NP,), jnp.float32),
            pltpu.VMEM((NP,), jnp.float32),
            pltpu.SemaphoreType.DMA,
        ],
        compiler_params=pltpu.CompilerParams(needs_layout_passes=False),
    )(src3d, dst3d)


def _agg_body(src_hbm, dst_hbm, xs_hbm, zrows_hbm, out_hbm,
              src_idx, dst_idx, rows, acc, sem):
    c = lax.axis_index("c")
    s = lax.axis_index("s")
    wid = c * NS + s
    rbase = s * ROWS_PER_TILE

    pltpu.sync_copy(src_hbm.at[wid], src_idx)
    pltpu.sync_copy(dst_hbm.at[wid], dst_idx)
    pltpu.sync_copy(zrows_hbm, acc.at[pl.ds(rbase, ROWS_PER_TILE)])
    plsc.subcore_barrier()

    def step(g, _):
        pltpu.async_copy(xs_hbm.at[src_idx.at[g]], rows, sem).wait()
        pltpu.sync_copy(rows, acc.at[dst_idx.at[g]], add=True)
        return 0

    lax.fori_loop(0, GMAX, step, 0)
    plsc.subcore_barrier()
    pltpu.sync_copy(acc.at[pl.ds(rbase, ROWS_PER_TILE)],
                    out_hbm.at[c, pl.ds(rbase, ROWS_PER_TILE)])


def _sc_aggregate(src3d, dst3d, xs, zrows):
    return pl.kernel(
        _agg_body,
        out_type=jax.ShapeDtypeStruct((NC, NP, D), jnp.float32),
        mesh=_sc_mesh(),
        scratch_types=[
            pltpu.VMEM((GMAX, CHUNK), jnp.int32),
            pltpu.VMEM((GMAX, CHUNK), jnp.int32),
            pltpu.VMEM((CHUNK, D), jnp.float32),
            pltpu.VMEM_SHARED((NP, D), jnp.float32),
            pltpu.SemaphoreType.DMA,
        ],
    )(src3d, dst3d, xs, zrows)


def _red_body(deg_ref, out_ref):
    out_ref[...] = jnp.sum(deg_ref[...], axis=0)


def _tc_reduce(deg):
    return pl.pallas_call(
        _red_body,
        in_specs=[pl.BlockSpec((NC * NS, 2, NP), lambda: (0, 0, 0))],
        out_specs=pl.BlockSpec((2, NP), lambda: (0, 0)),
        out_shape=jax.ShapeDtypeStruct((2, NP), jnp.float32),
    )(deg)


def _pre_body(deg_ref, h_ref, xs_ref, ns_ref, nd_ref):
    i = pl.program_id(0)
    d = deg_ref[...]
    row = i * R_BLK + lax.broadcasted_iota(jnp.int32, (R_BLK, 1), 0)
    valid = row < N
    ns = jnp.where(valid, lax.rsqrt(jnp.maximum(d[0], 1.0)), 0.0)
    nd = jnp.where(valid, lax.rsqrt(jnp.maximum(d[1], 1.0)), 0.0)
    ns_ref[...] = ns
    nd_ref[...] = nd
    xs_ref[...] = h_ref[...] * ns


def _tc_pre(deg3, h_pad):
    grid = NP // R_BLK
    return pl.pallas_call(
        _pre_body,
        grid=(grid,),
        in_specs=[
            pl.BlockSpec((2, R_BLK, 1), lambda i: (0, i, 0)),
            pl.BlockSpec((R_BLK, D), lambda i: (i, 0)),
        ],
        out_specs=[
            pl.BlockSpec((R_BLK, D), lambda i: (i, 0)),
            pl.BlockSpec((R_BLK, 1), lambda i: (i, 0)),
            pl.BlockSpec((R_BLK, 1), lambda i: (i, 0)),
        ],
        out_shape=[
            jax.ShapeDtypeStruct((NP, D), jnp.float32),
            jax.ShapeDtypeStruct((NP, 1), jnp.float32),
            jax.ShapeDtypeStruct((NP, 1), jnp.float32),
        ],
    )(deg3, h_pad)


def _layer_body(agg_ref, nd_ref, ns_ref, w_ref, b_ref, out_ref, *, relu, scale):
    a = agg_ref[0] + agg_ref[1]
    y = jnp.dot(a * nd_ref[...], w_ref[...],
                preferred_element_type=jnp.float32) + b_ref[...]
    if relu:
        y = jnp.maximum(y, 0.0)
    if scale:
        y = y * ns_ref[...]
    out_ref[...] = y


def _tc_layer(agg, nd, ns, w, b2d, relu, scale):
    grid = NP // R_BLK
    return pl.pallas_call(
        functools.partial(_layer_body, relu=relu, scale=scale),
        grid=(grid,),
        in_specs=[
            pl.BlockSpec((NC, R_BLK, D), lambda i: (0, i, 0)),
            pl.BlockSpec((R_BLK, 1), lambda i: (i, 0)),
            pl.BlockSpec((R_BLK, 1), lambda i: (i, 0)),
            pl.BlockSpec((D, D), lambda i: (0, 0)),
            pl.BlockSpec((1, D), lambda i: (0, 0)),
        ],
        out_specs=pl.BlockSpec((R_BLK, D), lambda i: (i, 0)),
        out_shape=jax.ShapeDtypeStruct((NP, D), jnp.float32),
    )(agg, nd, ns, w, b2d)


def kernel(h, edge_index, W1, b1, W2, b2, W3, b3):
    e32 = edge_index.astype(jnp.int32)
    pad = jnp.full((EPAD - E,), N, dtype=jnp.int32)
    src2d = jnp.concatenate([e32[0], pad]).reshape(NC * NS, GROUPS, CHUNK)
    dst2d = jnp.concatenate([e32[1], pad]).reshape(NC * NS, GROUPS, CHUNK)
    h_pad = jnp.zeros((NP, D), jnp.float32).at[:N].set(h)
    zrows = jnp.zeros((ROWS_PER_TILE, D), jnp.float32)

    deg = _tc_reduce(_sc_degrees(src2d, dst2d))
    xs, ns, nd = _tc_pre(deg.reshape(2, NP, 1), h_pad)
    agg = _sc_aggregate(src2d, dst2d, xs, zrows)
    xs = _tc_layer(agg, nd, ns, W1, b1.reshape(1, D), True, True)
    agg = _sc_aggregate(src2d, dst2d, xs, zrows)
    xs = _tc_layer(agg, nd, ns, W2, b2.reshape(1, D), True, True)
    agg = _sc_aggregate(src2d, dst2d, xs, zrows)
    out = _tc_layer(agg, nd, ns, W3, b3.reshape(1, D), False, False)
    return out[:N]
